# Initial kernel scaffold; baseline (speedup 1.0000x reference)
import jax
import jax.numpy as jnp
from jax.experimental import pallas as pl


def kernel(x, w_gate, w_noise):
    raise NotImplementedError("write your pallas kernel here")


# speedup vs baseline: 5.5047x; 5.5047x over previous
import functools

import jax
import jax.numpy as jnp
from jax.experimental import pallas as pl
from jax.experimental.pallas import tpu as pltpu

N = 8192
D = 4096
E = 64
K = 8
BN = 512


def _router_body(x_ref, w_ref, rout_ref, part_ref):
    logits = jnp.dot(x_ref[...], w_ref[...],
                     preferred_element_type=jnp.float32)
    rowmax = jnp.max(logits, axis=-1, keepdims=True)
    e = jnp.exp(logits - rowmax)
    p = e / jnp.sum(e, axis=-1, keepdims=True)
    part_ref[0, 0, 0] = jnp.sum(p)
    part_ref[0, 0, 1] = jnp.sum(p * p)

    m = logits
    t = rowmax
    for _ in range(K):
        t = jnp.max(m, axis=-1, keepdims=True)
        m = jnp.where(m >= t, -jnp.inf, m)
    mask = logits >= t
    es = jnp.where(mask, e, 0.0)
    rout_ref[...] = es / jnp.sum(es, axis=-1, keepdims=True)


@jax.jit
def kernel(x, w_gate, w_noise):
    del w_noise
    nblocks = N // BN
    router, partials = pl.pallas_call(
        _router_body,
        grid=(nblocks,),
        in_specs=[
            pl.BlockSpec((BN, D), lambda i: (i, 0)),
            pl.BlockSpec((D, E), lambda i: (0, 0)),
        ],
        out_specs=[
            pl.BlockSpec((BN, E), lambda i: (i, 0)),
            pl.BlockSpec((1, 1, 2), lambda i: (i, 0, 0),
                         memory_space=pltpu.SMEM),
        ],
        out_shape=[
            jax.ShapeDtypeStruct((N, E), jnp.float32),
            jax.ShapeDtypeStruct((nblocks, 1, 2), jnp.float32),
        ],
        compiler_params=pltpu.CompilerParams(
            dimension_semantics=("parallel",),
        ),
    )(x, w_gate)
    s1 = jnp.sum(partials[:, 0, 0])
    s2 = jnp.sum(partials[:, 0, 1])
    n = N * E
    mean = s1 / n
    var = (s2 - s1 * s1 / n) / (n - 1)
    load_loss = var / (mean * mean + 1e-10)
    return (router, load_loss)
